# baseline (speedup 1.0000x reference)
import functools

import jax
import jax.numpy as jnp
from jax import lax
from jax.experimental import pallas as pl
from jax.experimental.pallas import tpu as pltpu
from jax.experimental.pallas import tpu_sc as plsc

B = 16384
E = 10
S = 26
D = 16
TR = 32
BLK = 16384
CH = 16
NG = B // 128


@functools.lru_cache(maxsize=None)
def _make_sc_gather():
    info = plsc.get_sparse_core_info()
    nc, ns = info.num_cores, info.num_subcores
    nw = nc * ns
    bpw = B // nw
    gpw = bpw // 128
    mesh = plsc.VectorSubcoreMesh(core_axis_name="c", subcore_axis_name="s")

    @functools.partial(
        pl.kernel,
        mesh=mesh,
        out_type=jax.ShapeDtypeStruct((NG * D, 128), jnp.float32),
        scratch_types=[
            pltpu.VMEM((TR, D), jnp.float32),
            pltpu.VMEM((bpw,), jnp.int32),
            pltpu.VMEM((bpw,), jnp.float32),
            pltpu.VMEM((gpw * D, 128), jnp.float32),
            pltpu.SemaphoreType.DMA,
        ],
        compiler_params=pltpu.CompilerParams(
            use_tc_tiling_on_sc=False, needs_layout_passes=False),
    )
    def sc_gather(tbl_hbm, idx_hbm, dlt_hbm, out_hbm,
                  tbl_v, idx_v, dlt_v, gt_v, sem):
        wid = lax.axis_index("s") * nc + lax.axis_index("c")
        base = wid * bpw
        cp1 = pltpu.async_copy(tbl_hbm, tbl_v, sem)
        cp2 = pltpu.async_copy(idx_hbm.at[pl.ds(base, bpw)], idx_v, sem)
        cp3 = pltpu.async_copy(dlt_hbm.at[pl.ds(base, bpw)], dlt_v, sem)
        cp1.wait()
        cp2.wait()
        cp3.wait()

        def chunk(k, carry):
            c16 = idx_v[pl.ds(k * CH, CH)]
            row0 = D * (k // 8)
            lane0 = (k % 8) * CH
            for j in range(E):
                cj = jnp.full((CH,), j, jnp.int32)
                gt_v[row0 + j, pl.ds(lane0, CH)] = (
                    plsc.load_gather(tbl_v, [c16, cj]))
            gt_v[row0 + E, pl.ds(lane0, CH)] = dlt_v[pl.ds(k * CH, CH)]
            return carry

        lax.fori_loop(0, bpw // CH, chunk, 0, unroll=4)

        pltpu.sync_copy(gt_v, out_hbm.at[pl.ds(wid * gpw * D, gpw * D)])

    return sc_gather


def _tc_body(g_ref, w_ref, b_ref, o_ref):
    gt = jnp.concatenate(
        [g_ref[D * c:D * c + E + 1, :] for c in range(BLK // 128)],
        axis=1)
    logits = lax.dot_general(
        w_ref[...], gt, (((1,), (0,)), ((), ())),
        preferred_element_type=jnp.float32)
    logits = logits + b_ref[...]
    m = jnp.max(logits, axis=0, keepdims=True)
    e = jnp.exp(logits - m)
    s = jnp.sum(e, axis=0, keepdims=True)
    o_ref[...] = logits - m - jnp.log(s)


def kernel(c_idx, delta, emb_table, W, b):
    tbl = jnp.zeros((TR, D), jnp.float32).at[:S, :E].set(emb_table)
    g_s = _make_sc_gather()(tbl, c_idx.astype(jnp.int32), delta)

    out_t = pl.pallas_call(
        _tc_body,
        grid=(B // BLK,),
        in_specs=[
            pl.BlockSpec((BLK // 128 * D, 128), lambda i: (i, 0)),
            pl.BlockSpec((S, E + 1), lambda i: (0, 0)),
            pl.BlockSpec((S, 1), lambda i: (0, 0)),
        ],
        out_specs=pl.BlockSpec((S, BLK), lambda i: (0, i)),
        out_shape=jax.ShapeDtypeStruct((S, B), jnp.float32),
    )(g_s, W, b.reshape(S, 1))
    return out_t.T


# speedup vs baseline: 1.0120x; 1.0120x over previous
import functools

import jax
import jax.numpy as jnp
from jax import lax
from jax.experimental import pallas as pl
from jax.experimental.pallas import tpu as pltpu
from jax.experimental.pallas import tpu_sc as plsc

B = 16384
E = 10
S = 26
D = 16
TR = 32
BLK = 8192
CH = 16
NG = B // 128


@functools.lru_cache(maxsize=None)
def _make_sc_gather():
    info = plsc.get_sparse_core_info()
    nc, ns = info.num_cores, info.num_subcores
    nw = nc * ns
    bpw = B // nw
    gpw = bpw // 128
    mesh = plsc.VectorSubcoreMesh(core_axis_name="c", subcore_axis_name="s")

    @functools.partial(
        pl.kernel,
        mesh=mesh,
        out_type=jax.ShapeDtypeStruct((NG * D, 128), jnp.float32),
        scratch_types=[
            pltpu.VMEM((TR, D), jnp.float32),
            pltpu.VMEM((bpw,), jnp.int32),
            pltpu.VMEM((bpw,), jnp.float32),
            pltpu.VMEM((gpw * D, 128), jnp.float32),
            pltpu.SemaphoreType.DMA,
        ],
        compiler_params=pltpu.CompilerParams(
            use_tc_tiling_on_sc=False, needs_layout_passes=False),
    )
    def sc_gather(tbl_hbm, idx_hbm, dlt_hbm, out_hbm,
                  tbl_v, idx_v, dlt_v, gt_v, sem):
        wid = lax.axis_index("s") * nc + lax.axis_index("c")
        base = wid * bpw
        cp1 = pltpu.async_copy(tbl_hbm, tbl_v, sem)
        cp2 = pltpu.async_copy(idx_hbm.at[pl.ds(base, bpw)], idx_v, sem)
        cp3 = pltpu.async_copy(dlt_hbm.at[pl.ds(base, bpw)], dlt_v, sem)
        cp1.wait()
        cp2.wait()

        def chunk(k, carry):
            c16 = idx_v[pl.ds(k * CH, CH)]
            row0 = D * (k // 8)
            lane0 = (k % 8) * CH
            for j in range(E):
                cj = jnp.full((CH,), j, jnp.int32)
                gt_v[row0 + j, pl.ds(lane0, CH)] = (
                    plsc.load_gather(tbl_v, [c16, cj]))
            return carry

        lax.fori_loop(0, bpw // CH, chunk, 0, unroll=4)

        cp3.wait()

        def dchunk(k, carry):
            gt_v[D * (k // 8) + E, pl.ds((k % 8) * CH, CH)] = (
                dlt_v[pl.ds(k * CH, CH)])
            return carry

        lax.fori_loop(0, bpw // CH, dchunk, 0, unroll=8)

        pltpu.sync_copy(gt_v, out_hbm.at[pl.ds(wid * gpw * D, gpw * D)])

    return sc_gather


def _tc_body(g_ref, w_ref, b_ref, o_ref):
    gt = jnp.concatenate(
        [g_ref[D * c:D * c + E + 1, :] for c in range(BLK // 128)],
        axis=1)
    logits = lax.dot_general(
        w_ref[...], gt, (((1,), (0,)), ((), ())),
        preferred_element_type=jnp.float32)
    logits = logits + b_ref[...]
    m = jnp.max(logits, axis=0, keepdims=True)
    e = jnp.exp(logits - m)
    s = jnp.sum(e, axis=0, keepdims=True)
    o_ref[...] = logits - m - jnp.log(s)


def kernel(c_idx, delta, emb_table, W, b):
    tbl = jnp.zeros((TR, D), jnp.float32).at[:S, :E].set(emb_table)
    g_s = _make_sc_gather()(tbl, c_idx.astype(jnp.int32), delta)

    out_t = pl.pallas_call(
        _tc_body,
        grid=(B // BLK,),
        in_specs=[
            pl.BlockSpec((BLK // 128 * D, 128), lambda i: (i, 0)),
            pl.BlockSpec((S, E + 1), lambda i: (0, 0)),
            pl.BlockSpec((S, 1), lambda i: (0, 0)),
        ],
        out_specs=pl.BlockSpec((S, BLK), lambda i: (0, i)),
        out_shape=jax.ShapeDtypeStruct((S, B), jnp.float32),
    )(g_s, W, b.reshape(S, 1))
    return out_t.T


# speedup vs baseline: 1.0794x; 1.0666x over previous
import functools

import jax
import jax.numpy as jnp
from jax import lax
from jax.experimental import pallas as pl
from jax.experimental.pallas import tpu as pltpu
from jax.experimental.pallas import tpu_sc as plsc

B = 16384
E = 10
S = 26
D = 16
TR = 32
BLK = 8192
CH = 16
NG = B // 128


@functools.lru_cache(maxsize=None)
def _make_sc_gather():
    info = plsc.get_sparse_core_info()
    nc, ns = info.num_cores, info.num_subcores
    nw = nc * ns
    bpw = B // nw
    gpw = bpw // 128
    mesh = plsc.VectorSubcoreMesh(core_axis_name="c", subcore_axis_name="s")

    @functools.partial(
        pl.kernel,
        mesh=mesh,
        out_type=jax.ShapeDtypeStruct((NG * D, 128), jnp.float32),
        scratch_types=[
            pltpu.VMEM((TR, D), jnp.float32),
            pltpu.VMEM((bpw,), jnp.int32),
            pltpu.VMEM((bpw,), jnp.float32),
            pltpu.VMEM((gpw * D, 128), jnp.float32),
            pltpu.SemaphoreType.DMA,
        ],
        compiler_params=pltpu.CompilerParams(
            use_tc_tiling_on_sc=False, needs_layout_passes=False),
    )
    def sc_gather(tbl_hbm, idx_hbm, dlt_hbm, out_hbm,
                  tbl_v, idx_v, dlt_v, gt_v, sem):
        wid = lax.axis_index("s") * nc + lax.axis_index("c")
        base = wid * bpw
        cp1 = pltpu.async_copy(tbl_hbm, tbl_v, sem)
        cp2 = pltpu.async_copy(idx_hbm.at[pl.ds(base, bpw)], idx_v, sem)
        cp3 = pltpu.async_copy(dlt_hbm.at[pl.ds(base, bpw)], dlt_v, sem)
        cp1.wait()
        cp2.wait()

        @plsc.parallel_loop(0, bpw // CH, unroll=4)
        def chunk(k):
            c16 = idx_v[pl.ds(k * CH, CH)]
            row0 = D * (k // 8)
            lane0 = (k % 8) * CH
            for j in range(E):
                cj = jnp.full((CH,), j, jnp.int32)
                gt_v[row0 + j, pl.ds(lane0, CH)] = (
                    plsc.load_gather(tbl_v, [c16, cj]))

        cp3.wait()

        @plsc.parallel_loop(0, bpw // CH, unroll=8)
        def dchunk(k):
            gt_v[D * (k // 8) + E, pl.ds((k % 8) * CH, CH)] = (
                dlt_v[pl.ds(k * CH, CH)])

        pltpu.sync_copy(gt_v, out_hbm.at[pl.ds(wid * gpw * D, gpw * D)])

    return sc_gather


def _tc_body(g_ref, w_ref, b_ref, o_ref):
    gt = jnp.concatenate(
        [g_ref[D * c:D * c + E + 1, :] for c in range(BLK // 128)],
        axis=1)
    logits = lax.dot_general(
        w_ref[...], gt, (((1,), (0,)), ((), ())),
        preferred_element_type=jnp.float32)
    logits = logits + b_ref[...]
    m = jnp.max(logits, axis=0, keepdims=True)
    e = jnp.exp(logits - m)
    s = jnp.sum(e, axis=0, keepdims=True)
    o_ref[...] = logits - m - jnp.log(s)


def kernel(c_idx, delta, emb_table, W, b):
    tbl = jnp.zeros((TR, D), jnp.float32).at[:S, :E].set(emb_table)
    g_s = _make_sc_gather()(tbl, c_idx.astype(jnp.int32), delta)

    out_t = pl.pallas_call(
        _tc_body,
        grid=(B // BLK,),
        in_specs=[
            pl.BlockSpec((BLK // 128 * D, 128), lambda i: (i, 0)),
            pl.BlockSpec((S, E + 1), lambda i: (0, 0)),
            pl.BlockSpec((S, 1), lambda i: (0, 0)),
        ],
        out_specs=pl.BlockSpec((S, BLK), lambda i: (0, i)),
        out_shape=jax.ShapeDtypeStruct((S, B), jnp.float32),
    )(g_s, W, b.reshape(S, 1))
    return out_t.T
